# baseline (speedup 1.0000x reference)
import jax
import jax.numpy as jnp
from jax.experimental import pallas as pl

NUM_PARTITIONS = 8
TBLK = 256


def _mask_bcast_kernel(idx_ref, x_ref, out_ref):
    K = idx_ref.shape[1]
    piota = jax.lax.broadcasted_iota(jnp.int32, (TBLK, NUM_PARTITIONS, 1), 1)
    m = idx_ref[:, 0:1, :] == piota
    for k in range(1, K):
        m = m | (idx_ref[:, k:k + 1, :] == piota)
    out_ref[...] = jnp.where(m, x_ref[...], 0.0)


def kernel(x, partition_indices):
    B, S, D = x.shape
    T = B * S
    K = partition_indices.shape[-1]
    xf = x.reshape(T, 1, D)
    idx = partition_indices.reshape(T, K, 1).astype(jnp.int32)

    out = pl.pallas_call(
        _mask_bcast_kernel,
        grid=(T // TBLK,),
        in_specs=[
            pl.BlockSpec((TBLK, K, 1), lambda i: (i, 0, 0)),
            pl.BlockSpec((TBLK, 1, D), lambda i: (i, 0, 0)),
        ],
        out_specs=pl.BlockSpec((TBLK, NUM_PARTITIONS, D), lambda i: (i, 0, 0)),
        out_shape=jax.ShapeDtypeStruct((T, NUM_PARTITIONS, D), x.dtype),
    )(idx, xf)
    return out.reshape(B, S, NUM_PARTITIONS, D)


# speedup vs baseline: 1.0067x; 1.0067x over previous
import jax
import jax.numpy as jnp
from jax.experimental import pallas as pl

NUM_PARTITIONS = 8
TBLK = 512


def _mask_bcast_kernel(idx_ref, x_ref, out_ref):
    K = idx_ref.shape[1]
    piota = jax.lax.broadcasted_iota(jnp.int32, (TBLK, NUM_PARTITIONS, 1), 1)
    m = idx_ref[:, 0:1, :] == piota
    for k in range(1, K):
        m = m | (idx_ref[:, k:k + 1, :] == piota)
    out_ref[...] = jnp.where(m, x_ref[...], 0.0)


def kernel(x, partition_indices):
    B, S, D = x.shape
    T = B * S
    K = partition_indices.shape[-1]
    xf = x.reshape(T, 1, D)
    idx = partition_indices.reshape(T, K, 1).astype(jnp.int32)

    out = pl.pallas_call(
        _mask_bcast_kernel,
        grid=(T // TBLK,),
        in_specs=[
            pl.BlockSpec((TBLK, K, 1), lambda i: (i, 0, 0)),
            pl.BlockSpec((TBLK, 1, D), lambda i: (i, 0, 0)),
        ],
        out_specs=pl.BlockSpec((TBLK, NUM_PARTITIONS, D), lambda i: (i, 0, 0)),
        out_shape=jax.ShapeDtypeStruct((T, NUM_PARTITIONS, D), x.dtype),
    )(idx, xf)
    return out.reshape(B, S, NUM_PARTITIONS, D)
